# Initial kernel scaffold; baseline (speedup 1.0000x reference)
import jax
import jax.numpy as jnp
from jax.experimental import pallas as pl


def kernel(x, params, edge_index):
    raise NotImplementedError("write your pallas kernel here")


# speedup vs baseline: 1.2120x; 1.2120x over previous
import functools

import jax
import jax.numpy as jnp
from jax.experimental import pallas as pl
from jax.experimental.pallas import tpu as pltpu

_N = 10000
_E = 160000
_NHEADS = 4
_ALPHA = 0.01


def _mm_scores_body(x_ref, w_ref, a_ref, wh_ref, s_ref, *, elu_input):
    x = x_ref[...]
    if elu_input:
        x = jnp.where(x > 0, x, jnp.exp(x) - 1.0)
    wh = jnp.dot(x, w_ref[...], preferred_element_type=jnp.float32)
    wh_ref[...] = wh
    s_ref[...] = jnp.dot(wh, a_ref[...], preferred_element_type=jnp.float32)


def _mm_scores(x, W, A, block_rows=1000, elu_input=False):
    n, k = x.shape
    m = W.shape[1]
    c = A.shape[1]
    return pl.pallas_call(
        functools.partial(_mm_scores_body, elu_input=elu_input),
        grid=(n // block_rows,),
        in_specs=[
            pl.BlockSpec((block_rows, k), lambda i: (i, 0)),
            pl.BlockSpec((k, m), lambda i: (0, 0)),
            pl.BlockSpec((m, c), lambda i: (0, 0)),
        ],
        out_specs=[
            pl.BlockSpec((block_rows, m), lambda i: (i, 0)),
            pl.BlockSpec((block_rows, c), lambda i: (i, 0)),
        ],
        out_shape=[
            jax.ShapeDtypeStruct((n, m), jnp.float32),
            jax.ShapeDtypeStruct((n, c), jnp.float32),
        ],
    )(x, W, A)


def _mlp_body(emb_ref, w1_ref, b1_ref, g_ref, b_ref, w2_ref, b2_ref, out_ref):
    g = jnp.dot(emb_ref[...], w1_ref[...], preferred_element_type=jnp.float32)
    g = g + b1_ref[...]
    g = jnp.maximum(g, 0.0)
    mu = jnp.mean(g, axis=-1, keepdims=True)
    var = jnp.mean((g - mu) ** 2, axis=-1, keepdims=True)
    g = (g - mu) / jnp.sqrt(var + 1e-5) * g_ref[...] + b_ref[...]
    out_ref[...] = jnp.dot(g, w2_ref[...], preferred_element_type=jnp.float32) + b2_ref[...]


def _mlp(emb, w1, b1, ln_g, ln_b, w2, b2, block_rows=1000):
    n, k = emb.shape
    h = w1.shape[1]
    m = w2.shape[1]
    b1 = b1.reshape(1, h)
    ln_g = ln_g.reshape(1, h)
    ln_b = ln_b.reshape(1, h)
    b2 = b2.reshape(1, m)
    return pl.pallas_call(
        _mlp_body,
        grid=(n // block_rows,),
        in_specs=[
            pl.BlockSpec((block_rows, k), lambda i: (i, 0)),
            pl.BlockSpec((k, h), lambda i: (0, 0)),
            pl.BlockSpec((1, h), lambda i: (0, 0)),
            pl.BlockSpec((1, h), lambda i: (0, 0)),
            pl.BlockSpec((1, h), lambda i: (0, 0)),
            pl.BlockSpec((h, m), lambda i: (0, 0)),
            pl.BlockSpec((1, m), lambda i: (0, 0)),
        ],
        out_specs=pl.BlockSpec((block_rows, m), lambda i: (i, 0)),
        out_shape=jax.ShapeDtypeStruct((n, m), jnp.float32),
    )(emb, w1, b1, ln_g, ln_b, w2, b2)


def _edge_softmax_matvec(wh, s, d, src, dst):
    e = s[src] + d[dst]
    e = jnp.where(e > 0, e, _ALPHA * e)
    att = jnp.exp(e - jnp.max(e, axis=0))
    att_sum = jax.ops.segment_sum(att, dst, num_segments=_N)
    att = att / (att_sum[dst] + 1e-9)
    hdim = wh.shape[1] // s.shape[1]
    w = jnp.repeat(att, hdim, axis=1)
    return jax.ops.segment_sum(w * wh[src], dst, num_segments=_N)


def kernel(x, params, edge_index):
    src = edge_index[0]
    dst = edge_index[1]

    W1 = jnp.concatenate([params['W%d' % i] for i in range(_NHEADS)], axis=1)
    nhid = params['W0'].shape[1]
    blocks = []
    for i in range(_NHEADS):
        col_s = jnp.zeros((nhid, _NHEADS), jnp.float32).at[:, i].set(params['a_src%d' % i][:, 0])
        col_d = jnp.zeros((nhid, _NHEADS), jnp.float32).at[:, i].set(params['a_dst%d' % i][:, 0])
        blocks.append(jnp.concatenate([col_s, col_d], axis=1))
    A1 = jnp.concatenate(blocks, axis=0)
    wh1, sd1 = _mm_scores(x, W1, A1)
    s1, d1 = sd1[:, :_NHEADS], sd1[:, _NHEADS:]
    raw1 = _edge_softmax_matvec(wh1, s1, d1, src, dst)

    A2 = jnp.concatenate([params['a_src_out'], params['a_dst_out']], axis=1)
    wh2, sd2 = _mm_scores(raw1, params['W_out'], A2, elu_input=True)
    emb = _edge_softmax_matvec(wh2, sd2[:, :1], sd2[:, 1:], src, dst)

    gene = _mlp(emb, params['g1W'], params['g1b'], params['ln_g'], params['ln_b'],
                params['g2W'], params['g2b'])
    return (emb, gene)
